# baseline (speedup 1.0000x reference)
import functools

import jax
import jax.numpy as jnp
import numpy as np
from jax import lax
from jax.experimental import pallas as pl
from jax.experimental.pallas import tpu as pltpu
from jax.experimental.pallas import tpu_sc as plsc

BATCH = 1024
SEQ = 200
D = 128
NUM_CLASS = 4
NUM_SNV = 5
NUM_CHROM = 25
MAX_LEN = 5000
TOKENS = BATCH * SEQ

NUM_CORES = 2
NUM_SUBCORES = 16
NW = NUM_CORES * NUM_SUBCORES
PER_W = TOKENS // NW
CHUNK = 128
NCHUNK = PER_W // CHUNK
NROWS2D = TOKENS // CHUNK
NB = 3

FUSED_ROWS = 512


def _pos_table() -> jnp.ndarray:
    pos = np.arange(MAX_LEN, dtype=np.float32).reshape(-1, 1)
    denoms = np.power(10000.0, np.arange(0, D, 2, dtype=np.float32) / D)
    x = pos / denoms
    p = np.zeros((MAX_LEN, D), dtype=np.float32)
    p[:, 0::2] = np.sin(x)
    p[:, 1::2] = np.cos(x)
    return jnp.asarray(p)


def _fused_table_body(snv_ref, chrom_ref, out_ref):
    r = lax.broadcasted_iota(jnp.int32, (FUSED_ROWS, D), 0)
    col = lax.broadcasted_iota(jnp.int32, (FUSED_ROWS, D), 1)
    c = r // 125
    cls_part = jnp.where((col == c) & (r < 500), 1.0, 0.0).astype(jnp.float32)

    r8 = lax.broadcasted_iota(jnp.int32, (FUSED_ROWS, 8), 0)
    s8 = (r8 // 25) % 5
    oh_s = (lax.broadcasted_iota(jnp.int32, (FUSED_ROWS, 8), 1) == s8)
    snv_part = jnp.dot(oh_s.astype(jnp.float32), snv_ref[...],
                       preferred_element_type=jnp.float32)

    r32 = lax.broadcasted_iota(jnp.int32, (FUSED_ROWS, 32), 0)
    ch32 = r32 % 25
    oh_ch = (lax.broadcasted_iota(jnp.int32, (FUSED_ROWS, 32), 1) == ch32)
    chrom_part = jnp.dot(oh_ch.astype(jnp.float32), chrom_ref[...],
                         preferred_element_type=jnp.float32)

    out_ref[...] = cls_part + snv_part + chrom_part


def _build_fused_table(snv_table, chrom_table):
    snv_pad = jnp.zeros((8, D), jnp.float32).at[0:NUM_SNV, 4:124].set(snv_table)
    chrom_pad = jnp.zeros((32, D), jnp.float32).at[0:NUM_CHROM, 124:128].set(
        chrom_table)
    return pl.pallas_call(
        _fused_table_body,
        out_shape=jax.ShapeDtypeStruct((FUSED_ROWS, D), jnp.float32),
    )(snv_pad, chrom_pad)


def _sc_body(p_hbm, tab_hbm, pos_hbm, cls_hbm, snv_hbm, chr_hbm, out_hbm,
             pos2, fused2, p_sp, tab_sp, *rest):
    wid = lax.axis_index("s") * NUM_CORES + lax.axis_index("c")
    sid = lax.axis_index("s")
    base0 = wid * PER_W

    bufs = rest[0:NB]
    psems = rest[NB:2 * NB]
    tsems = rest[2 * NB:3 * NB]
    osems = rest[3 * NB:4 * NB]

    @pl.when(sid == 0)
    def _():
        pltpu.sync_copy(p_hbm, p_sp)
        pltpu.sync_copy(tab_hbm, tab_sp)

    pltpu.sync_copy(pos_hbm.at[wid], pos2)

    def prolog(cls_t, snv_t, chr_t):
        pltpu.sync_copy(cls_hbm.at[wid], cls_t)
        pltpu.sync_copy(snv_hbm.at[wid], snv_t)
        pltpu.sync_copy(chr_hbm.at[wid], chr_t)

        def frow(g, c):
            for i in range(CHUNK // 16):
                sl = pl.ds(i * 16, 16)
                fused2[g, sl] = cls_t[g, sl] * 125 + snv_t[g, sl] * 25 + chr_t[g, sl]
            return c

        lax.fori_loop(0, NCHUNK, frow, 0)

    pl.run_scoped(
        prolog,
        pltpu.VMEM((NCHUNK, CHUNK), jnp.int32),
        pltpu.VMEM((NCHUNK, CHUNK), jnp.int32),
        pltpu.VMEM((NCHUNK, CHUNK), jnp.int32),
    )

    plsc.subcore_barrier()

    pcp = [None] * NCHUNK
    tcp = [None] * NCHUNK
    scp = [None] * NCHUNK
    for step in range(NCHUNK + 2):
        g = step
        if g < NCHUNK:
            b = g % NB
            if g >= NB:
                scp[g - NB].wait()
            p_src = p_hbm if g % 8 == 7 else p_sp
            pcp[g] = pltpu.async_copy(p_src.at[pos2.at[g]], bufs[b], psems[b])
        ga = step - 1
        if 0 <= ga < NCHUNK:
            b = ga % NB
            pcp[ga].wait()
            t_src = tab_hbm if ga % 8 == 7 else tab_sp
            tcp[ga] = pltpu.async_copy(t_src.at[fused2.at[ga]], bufs[b],
                                       tsems[b], add=True)
        gs = step - 2
        if 0 <= gs < NCHUNK:
            b = gs % NB
            tcp[gs].wait()
            scp[gs] = pltpu.async_copy(
                bufs[b], out_hbm.at[pl.ds(base0 + gs * CHUNK, CHUNK)], osems[b])

    for g in range(NCHUNK - NB, NCHUNK):
        scp[g].wait()


@jax.jit
def _run(p, tab, pos, cls, snv, chrom):
    mesh = plsc.VectorSubcoreMesh(core_axis_name="c", subcore_axis_name="s")
    f = pl.kernel(
        _sc_body,
        mesh=mesh,
        out_type=jax.ShapeDtypeStruct((TOKENS, D), jnp.float32),
        scratch_types=[
            pltpu.VMEM((NCHUNK, CHUNK), jnp.int32),
            pltpu.VMEM((NCHUNK, CHUNK), jnp.int32),
            pltpu.VMEM_SHARED((MAX_LEN, D), jnp.float32),
            pltpu.VMEM_SHARED((FUSED_ROWS, D), jnp.float32),
        ] + [pltpu.VMEM((CHUNK, D), jnp.float32)] * NB
          + [pltpu.SemaphoreType.DMA] * (3 * NB),
    )
    return f(p, tab, pos, cls, snv, chrom)


def kernel(snvs, chromosomes, positions, class_ids, snv_table, chrom_table):
    tab = _build_fused_table(snv_table.astype(jnp.float32),
                             chrom_table.astype(jnp.float32))
    p = _pos_table()
    pos = positions.reshape(NW, NCHUNK, CHUNK).astype(jnp.int32)
    cls = class_ids.reshape(NW, NCHUNK, CHUNK).astype(jnp.int32)
    snv = snvs.reshape(NW, NCHUNK, CHUNK).astype(jnp.int32)
    chrom = chromosomes.reshape(NW, NCHUNK, CHUNK).astype(jnp.int32)
    out = _run(p, tab, pos, cls, snv, chrom)
    return out.reshape(BATCH, SEQ, D)


# speedup vs baseline: 1.0167x; 1.0167x over previous
import jax
import jax.numpy as jnp
import numpy as np
from jax import lax
from jax.experimental import pallas as pl
from jax.experimental.pallas import tpu as pltpu
from jax.experimental.pallas import tpu_sc as plsc

BATCH = 1024
SEQ = 200
D = 128
NUM_CLASS = 4
NUM_SNV = 5
NUM_CHROM = 25
MAX_LEN = 5000
TOKENS = BATCH * SEQ

NUM_CORES = 2
NUM_SUBCORES = 16
NW = NUM_CORES * NUM_SUBCORES
PER_W = TOKENS // NW
CHUNK = 128
NCHUNK = PER_W // CHUNK
NB = 3

FUSED_ROWS = 512


def _pos_table() -> jnp.ndarray:
    pos = np.arange(MAX_LEN, dtype=np.float32).reshape(-1, 1)
    denoms = np.power(10000.0, np.arange(0, D, 2, dtype=np.float32) / D)
    x = pos / denoms
    p = np.zeros((MAX_LEN, D), dtype=np.float32)
    p[:, 0::2] = np.sin(x)
    p[:, 1::2] = np.cos(x)
    return jnp.asarray(p)


def _fused_table_body(snv_ref, chrom_ref, out_ref):
    r = lax.broadcasted_iota(jnp.int32, (FUSED_ROWS, D), 0)
    col = lax.broadcasted_iota(jnp.int32, (FUSED_ROWS, D), 1)
    c = r // 125
    cls_part = jnp.where((col == c) & (r < 500), 1.0, 0.0).astype(jnp.float32)

    r8 = lax.broadcasted_iota(jnp.int32, (FUSED_ROWS, 8), 0)
    s8 = (r8 // 25) % 5
    oh_s = (lax.broadcasted_iota(jnp.int32, (FUSED_ROWS, 8), 1) == s8)
    snv_part = jnp.dot(oh_s.astype(jnp.float32), snv_ref[...],
                       preferred_element_type=jnp.float32)

    r32 = lax.broadcasted_iota(jnp.int32, (FUSED_ROWS, 32), 0)
    ch32 = r32 % 25
    oh_ch = (lax.broadcasted_iota(jnp.int32, (FUSED_ROWS, 32), 1) == ch32)
    chrom_part = jnp.dot(oh_ch.astype(jnp.float32), chrom_ref[...],
                         preferred_element_type=jnp.float32)

    out_ref[...] = cls_part + snv_part + chrom_part


def _build_fused_table(snv_table, chrom_table):
    snv_pad = jnp.zeros((8, D), jnp.float32).at[0:NUM_SNV, 4:124].set(snv_table)
    chrom_pad = jnp.zeros((32, D), jnp.float32).at[0:NUM_CHROM, 124:128].set(
        chrom_table)
    return pl.pallas_call(
        _fused_table_body,
        out_shape=jax.ShapeDtypeStruct((FUSED_ROWS, D), jnp.float32),
    )(snv_pad, chrom_pad)


def _sc_body(p_hbm, tab_hbm, idx_hbm, out_hbm,
             pos2, fused2, p_sp, tab_sp, *rest):
    wid = lax.axis_index("s") * NUM_CORES + lax.axis_index("c")
    sid = lax.axis_index("s")
    base0 = wid * PER_W

    bufs = rest[0:NB]
    psems = rest[NB:2 * NB]
    tsems = rest[2 * NB:3 * NB]
    osems = rest[3 * NB:4 * NB]

    @pl.when(sid == 0)
    def _():
        pltpu.sync_copy(p_hbm, p_sp)
        pltpu.sync_copy(tab_hbm, tab_sp)

    pltpu.sync_copy(idx_hbm.at[0, wid], pos2)

    def prolog(cls_t, snv_t, chr_t):
        pltpu.sync_copy(idx_hbm.at[1, wid], cls_t)
        pltpu.sync_copy(idx_hbm.at[2, wid], snv_t)
        pltpu.sync_copy(idx_hbm.at[3, wid], chr_t)

        def frow(g, c):
            for i in range(CHUNK // 16):
                sl = pl.ds(i * 16, 16)
                fused2[g, sl] = cls_t[g, sl] * 125 + snv_t[g, sl] * 25 + chr_t[g, sl]
            return c

        lax.fori_loop(0, NCHUNK, frow, 0)

    pl.run_scoped(
        prolog,
        pltpu.VMEM((NCHUNK, CHUNK), jnp.int32),
        pltpu.VMEM((NCHUNK, CHUNK), jnp.int32),
        pltpu.VMEM((NCHUNK, CHUNK), jnp.int32),
    )

    plsc.subcore_barrier()

    pcp = [None] * NCHUNK
    tcp = [None] * NCHUNK
    scp = [None] * NCHUNK
    for step in range(NCHUNK + 2):
        g = step
        if g < NCHUNK:
            b = g % NB
            if g >= NB:
                scp[g - NB].wait()
            pcp[g] = pltpu.async_copy(p_sp.at[pos2.at[g]], bufs[b], psems[b])
        ga = step - 1
        if 0 <= ga < NCHUNK:
            b = ga % NB
            pcp[ga].wait()
            tcp[ga] = pltpu.async_copy(tab_sp.at[fused2.at[ga]], bufs[b],
                                       tsems[b], add=True)
        gs = step - 2
        if 0 <= gs < NCHUNK:
            b = gs % NB
            tcp[gs].wait()
            scp[gs] = pltpu.async_copy(
                bufs[b], out_hbm.at[pl.ds(base0 + gs * CHUNK, CHUNK)], osems[b])

    for g in range(NCHUNK - NB, NCHUNK):
        scp[g].wait()


@jax.jit
def _run(p, tab, idx4):
    mesh = plsc.VectorSubcoreMesh(core_axis_name="c", subcore_axis_name="s")
    f = pl.kernel(
        _sc_body,
        mesh=mesh,
        out_type=jax.ShapeDtypeStruct((TOKENS, D), jnp.float32),
        scratch_types=[
            pltpu.VMEM((NCHUNK, CHUNK), jnp.int32),
            pltpu.VMEM((NCHUNK, CHUNK), jnp.int32),
            pltpu.VMEM_SHARED((MAX_LEN, D), jnp.float32),
            pltpu.VMEM_SHARED((FUSED_ROWS, D), jnp.float32),
        ] + [pltpu.VMEM((CHUNK, D), jnp.float32)] * NB
          + [pltpu.SemaphoreType.DMA] * (3 * NB),
    )
    return f(p, tab, idx4)


def kernel(snvs, chromosomes, positions, class_ids, snv_table, chrom_table):
    tab = _build_fused_table(snv_table.astype(jnp.float32),
                             chrom_table.astype(jnp.float32))
    p = _pos_table()
    idx4 = jnp.stack([positions.astype(jnp.int32), class_ids.astype(jnp.int32),
                      snvs.astype(jnp.int32), chromosomes.astype(jnp.int32)]
                     ).reshape(4, NW, NCHUNK, CHUNK)
    out = _run(p, tab, idx4)
    return out.reshape(BATCH, SEQ, D)


# speedup vs baseline: 1.0187x; 1.0019x over previous
import jax
import jax.numpy as jnp
import numpy as np
from jax import lax
from jax.experimental import pallas as pl
from jax.experimental.pallas import tpu as pltpu
from jax.experimental.pallas import tpu_sc as plsc

BATCH = 1024
SEQ = 200
D = 128
NUM_CLASS = 4
NUM_SNV = 5
NUM_CHROM = 25
MAX_LEN = 5000
TOKENS = BATCH * SEQ

NUM_CORES = 2
NUM_SUBCORES = 16
NW = NUM_CORES * NUM_SUBCORES
PER_W = TOKENS // NW
CHUNK = 128
NCHUNK = PER_W // CHUNK
NB = 3

FUSED_ROWS = 512
ROWS_PER_SUB = FUSED_ROWS // NUM_SUBCORES


def _pos_table() -> jnp.ndarray:
    pos = np.arange(MAX_LEN, dtype=np.float32).reshape(-1, 1)
    denoms = np.power(10000.0, np.arange(0, D, 2, dtype=np.float32) / D)
    x = pos / denoms
    p = np.zeros((MAX_LEN, D), dtype=np.float32)
    p[:, 0::2] = np.sin(x)
    p[:, 1::2] = np.cos(x)
    return jnp.asarray(p)


def _sc_body(p_hbm, snv_hbm, chr_hbm, idx_hbm, out_hbm,
             pos2, fused2, tabrows, p_sp, tab_sp, *rest):
    wid = lax.axis_index("s") * NUM_CORES + lax.axis_index("c")
    sid = lax.axis_index("s")
    base0 = wid * PER_W

    bufs = rest[0:NB]
    psems = rest[NB:2 * NB]
    tsems = rest[2 * NB:3 * NB]
    osems = rest[3 * NB:4 * NB]

    @pl.when(sid == 0)
    def _():
        pltpu.sync_copy(p_hbm, p_sp)

    def build_tab(snv_v, chr_v):
        pltpu.sync_copy(snv_hbm, snv_v)
        pltpu.sync_copy(chr_hbm, chr_v)
        r0 = sid * ROWS_PER_SUB

        def brow(i, c):
            r = r0 + i
            c_id = r // 125
            s_id = (r // 25) % 5
            ch_id = r % 25
            for j in range(D // 16):
                sl = pl.ds(j * 16, 16)
                v = snv_v[s_id, sl] + chr_v[ch_id, sl]
                if j == 0:
                    one = jnp.where(lax.iota(jnp.int32, 16) == c_id, 1.0, 0.0)
                    v = v + one.astype(jnp.float32)
                tabrows[i, sl] = v
            return c

        lax.fori_loop(0, ROWS_PER_SUB, brow, 0)
        pltpu.sync_copy(tabrows, tab_sp.at[pl.ds(sid * ROWS_PER_SUB,
                                                 ROWS_PER_SUB)])

    pl.run_scoped(
        build_tab,
        pltpu.VMEM((8, D), jnp.float32),
        pltpu.VMEM((32, D), jnp.float32),
    )

    pltpu.sync_copy(idx_hbm.at[0, wid], pos2)

    def prolog(cls_t, snv_t, chr_t):
        pltpu.sync_copy(idx_hbm.at[1, wid], cls_t)
        pltpu.sync_copy(idx_hbm.at[2, wid], snv_t)
        pltpu.sync_copy(idx_hbm.at[3, wid], chr_t)

        def frow(g, c):
            for i in range(CHUNK // 16):
                sl = pl.ds(i * 16, 16)
                fused2[g, sl] = cls_t[g, sl] * 125 + snv_t[g, sl] * 25 + chr_t[g, sl]
            return c

        lax.fori_loop(0, NCHUNK, frow, 0)

    pl.run_scoped(
        prolog,
        pltpu.VMEM((NCHUNK, CHUNK), jnp.int32),
        pltpu.VMEM((NCHUNK, CHUNK), jnp.int32),
        pltpu.VMEM((NCHUNK, CHUNK), jnp.int32),
    )

    plsc.subcore_barrier()

    pcp = [None] * NCHUNK
    tcp = [None] * NCHUNK
    scp = [None] * NCHUNK
    for step in range(NCHUNK + 2):
        g = step
        if g < NCHUNK:
            b = g % NB
            if g >= NB:
                scp[g - NB].wait()
            pcp[g] = pltpu.async_copy(p_sp.at[pos2.at[g]], bufs[b], psems[b])
        ga = step - 1
        if 0 <= ga < NCHUNK:
            b = ga % NB
            pcp[ga].wait()
            tcp[ga] = pltpu.async_copy(tab_sp.at[fused2.at[ga]], bufs[b],
                                       tsems[b], add=True)
        gs = step - 2
        if 0 <= gs < NCHUNK:
            b = gs % NB
            tcp[gs].wait()
            scp[gs] = pltpu.async_copy(
                bufs[b], out_hbm.at[pl.ds(base0 + gs * CHUNK, CHUNK)], osems[b])

    for g in range(NCHUNK - NB, NCHUNK):
        scp[g].wait()


@jax.jit
def _run(p, snv_pad, chrom_pad, idx4):
    mesh = plsc.VectorSubcoreMesh(core_axis_name="c", subcore_axis_name="s")
    f = pl.kernel(
        _sc_body,
        mesh=mesh,
        out_type=jax.ShapeDtypeStruct((TOKENS, D), jnp.float32),
        scratch_types=[
            pltpu.VMEM((NCHUNK, CHUNK), jnp.int32),
            pltpu.VMEM((NCHUNK, CHUNK), jnp.int32),
            pltpu.VMEM((ROWS_PER_SUB, D), jnp.float32),
            pltpu.VMEM_SHARED((MAX_LEN, D), jnp.float32),
            pltpu.VMEM_SHARED((FUSED_ROWS, D), jnp.float32),
        ] + [pltpu.VMEM((CHUNK, D), jnp.float32)] * NB
          + [pltpu.SemaphoreType.DMA] * (3 * NB),
    )
    return f(p, snv_pad, chrom_pad, idx4)


def kernel(snvs, chromosomes, positions, class_ids, snv_table, chrom_table):
    snv_pad = jnp.zeros((8, D), jnp.float32).at[0:NUM_SNV, 4:124].set(
        snv_table.astype(jnp.float32))
    chrom_pad = jnp.zeros((32, D), jnp.float32).at[0:NUM_CHROM, 124:128].set(
        chrom_table.astype(jnp.float32))
    p = _pos_table()
    idx4 = jnp.stack([positions.astype(jnp.int32), class_ids.astype(jnp.int32),
                      snvs.astype(jnp.int32), chromosomes.astype(jnp.int32)]
                     ).reshape(4, NW, NCHUNK, CHUNK)
    out = _run(p, snv_pad, chrom_pad, idx4)
    return out.reshape(BATCH, SEQ, D)


# speedup vs baseline: 1.0200x; 1.0014x over previous
import jax
import jax.numpy as jnp
import numpy as np
from jax import lax
from jax.experimental import pallas as pl
from jax.experimental.pallas import tpu as pltpu
from jax.experimental.pallas import tpu_sc as plsc

BATCH = 1024
SEQ = 200
D = 128
NUM_CLASS = 4
NUM_SNV = 5
NUM_CHROM = 25
MAX_LEN = 5000
TOKENS = BATCH * SEQ

NUM_CORES = 2
NUM_SUBCORES = 16
NW = NUM_CORES * NUM_SUBCORES
PER_W = TOKENS // NW
CHUNK = 128
NCHUNK = PER_W // CHUNK
NB = 3

FUSED_ROWS = 512
ROWS_PER_SUB = FUSED_ROWS // NUM_SUBCORES


def _pos_table() -> jnp.ndarray:
    pos = np.arange(MAX_LEN, dtype=np.float32).reshape(-1, 1)
    denoms = np.power(10000.0, np.arange(0, D, 2, dtype=np.float32) / D)
    x = pos / denoms
    p = np.zeros((MAX_LEN, D), dtype=np.float32)
    p[:, 0::2] = np.sin(x)
    p[:, 1::2] = np.cos(x)
    return jnp.asarray(p)


def _sc_body(p_hbm, snv_hbm, chr_hbm, idx_hbm, out_hbm,
             pos2, fused2, tabrows, p_sp, tab_sp, *rest):
    wid = lax.axis_index("s") * NUM_CORES + lax.axis_index("c")
    sid = lax.axis_index("s")
    base0 = wid * PER_W

    bufs = rest[0:NB]
    psems = rest[NB:2 * NB]
    tsems = rest[2 * NB:3 * NB]
    osems = rest[3 * NB:4 * NB]

    @pl.when(sid == 0)
    def _():
        pltpu.sync_copy(p_hbm, p_sp)

    def build_tab(snv_v, chr_v):
        pltpu.sync_copy(snv_hbm, snv_v)
        pltpu.sync_copy(chr_hbm, chr_v)
        r0 = sid * ROWS_PER_SUB

        def brow(i, c):
            r = r0 + i
            c_id = r // 125
            s_id = (r // 25) % 5
            ch_id = r % 25
            for j in range(D // 16):
                sl = pl.ds(j * 16, 16)
                v = snv_v[s_id, sl] + chr_v[ch_id, sl]
                if j == 0:
                    one = jnp.where(lax.iota(jnp.int32, 16) == c_id, 1.0, 0.0)
                    v = v + one.astype(jnp.float32)
                tabrows[i, sl] = v
            return c

        lax.fori_loop(0, ROWS_PER_SUB, brow, 0)
        pltpu.sync_copy(tabrows, tab_sp.at[pl.ds(sid * ROWS_PER_SUB,
                                                 ROWS_PER_SUB)])

    pl.run_scoped(
        build_tab,
        pltpu.VMEM((8, D), jnp.float32),
        pltpu.VMEM((32, D), jnp.float32),
    )

    pltpu.sync_copy(idx_hbm.at[0, wid], pos2)

    def prolog(cls_t, snv_t, chr_t):
        pltpu.sync_copy(idx_hbm.at[1, wid], cls_t)
        pltpu.sync_copy(idx_hbm.at[2, wid], snv_t)
        pltpu.sync_copy(idx_hbm.at[3, wid], chr_t)

        def frow(g, c):
            for i in range(CHUNK // 16):
                sl = pl.ds(i * 16, 16)
                fused2[g, sl] = cls_t[g, sl] * 125 + snv_t[g, sl] * 25 + chr_t[g, sl]
            return c

        lax.fori_loop(0, NCHUNK, frow, 0)

    pl.run_scoped(
        prolog,
        pltpu.VMEM((NCHUNK, CHUNK), jnp.int32),
        pltpu.VMEM((NCHUNK, CHUNK), jnp.int32),
        pltpu.VMEM((NCHUNK, CHUNK), jnp.int32),
    )

    plsc.subcore_barrier()

    pcp = [None] * NCHUNK
    tcp = [None] * NCHUNK
    scp = [None] * NCHUNK
    for step in range(NCHUNK + 2):
        g = step
        if g < NCHUNK:
            b = g % NB
            if g >= NB:
                scp[g - NB].wait()
            p_src = p_hbm if g % 4 == 3 else p_sp
            pcp[g] = pltpu.async_copy(p_src.at[pos2.at[g]], bufs[b], psems[b])
        ga = step - 1
        if 0 <= ga < NCHUNK:
            b = ga % NB
            pcp[ga].wait()
            tcp[ga] = pltpu.async_copy(tab_sp.at[fused2.at[ga]], bufs[b],
                                       tsems[b], add=True)
        gs = step - 2
        if 0 <= gs < NCHUNK:
            b = gs % NB
            tcp[gs].wait()
            scp[gs] = pltpu.async_copy(
                bufs[b], out_hbm.at[pl.ds(base0 + gs * CHUNK, CHUNK)], osems[b])

    for g in range(NCHUNK - NB, NCHUNK):
        scp[g].wait()


@jax.jit
def _run(p, snv_pad, chrom_pad, idx4):
    mesh = plsc.VectorSubcoreMesh(core_axis_name="c", subcore_axis_name="s")
    f = pl.kernel(
        _sc_body,
        mesh=mesh,
        out_type=jax.ShapeDtypeStruct((TOKENS, D), jnp.float32),
        scratch_types=[
            pltpu.VMEM((NCHUNK, CHUNK), jnp.int32),
            pltpu.VMEM((NCHUNK, CHUNK), jnp.int32),
            pltpu.VMEM((ROWS_PER_SUB, D), jnp.float32),
            pltpu.VMEM_SHARED((MAX_LEN, D), jnp.float32),
            pltpu.VMEM_SHARED((FUSED_ROWS, D), jnp.float32),
        ] + [pltpu.VMEM((CHUNK, D), jnp.float32)] * NB
          + [pltpu.SemaphoreType.DMA] * (3 * NB),
    )
    return f(p, snv_pad, chrom_pad, idx4)


def kernel(snvs, chromosomes, positions, class_ids, snv_table, chrom_table):
    snv_pad = jnp.zeros((8, D), jnp.float32).at[0:NUM_SNV, 4:124].set(
        snv_table.astype(jnp.float32))
    chrom_pad = jnp.zeros((32, D), jnp.float32).at[0:NUM_CHROM, 124:128].set(
        chrom_table.astype(jnp.float32))
    p = _pos_table()
    idx4 = jnp.stack([positions.astype(jnp.int32), class_ids.astype(jnp.int32),
                      snvs.astype(jnp.int32), chromosomes.astype(jnp.int32)]
                     ).reshape(4, NW, NCHUNK, CHUNK)
    out = _run(p, snv_pad, chrom_pad, idx4)
    return out.reshape(BATCH, SEQ, D)
